# Initial kernel scaffold; baseline (speedup 1.0000x reference)
import jax
import jax.numpy as jnp
from jax.experimental import pallas as pl


def kernel(x, Wr, br, gamma, beta, W1, b1, W2, b2):
    raise NotImplementedError("write your pallas kernel here")


# speedup vs baseline: 6.7298x; 6.7298x over previous
import functools

import jax
import jax.numpy as jnp
from jax import lax
from jax.experimental import pallas as pl
from jax.experimental.pallas import tpu as pltpu

E = 64
D = 768
H = 128
C = 128
B = 2048
LN_EPS = 1e-5

_F32 = jnp.float32
_HI = lax.Precision.DEFAULT


def _gelu(v):
    return 0.5 * v * (1.0 + lax.erf(v * 0.7071067811865476))


def _dense_body(x_ref, Wr_ref, br_ref, gamma_ref, beta_ref, W1_ref, b1_ref,
                W2_ref, b2_ref, out_ref, eid_ref, xhat_ref):
    j = pl.program_id(0)

    @pl.when(j == 0)
    def _init():
        logits = lax.dot_general(x_ref[...], Wr_ref[...],
                                 (((1,), (1,)), ((), ())),
                                 precision=_HI,
                                 preferred_element_type=_F32) + br_ref[...]
        eid_ref[...] = jnp.argmax(logits, axis=1).astype(jnp.int32)[:, None]
        xv = x_ref[...]
        mu = jnp.mean(xv, axis=1, keepdims=True)
        var = jnp.mean((xv - mu) ** 2, axis=1, keepdims=True)
        xhat_ref[...] = (xv - mu) * lax.rsqrt(var + LN_EPS)

    xn = xhat_ref[...] * gamma_ref[0] + beta_ref[0]
    h = _gelu(lax.dot_general(xn, W1_ref[0], (((1,), (0,)), ((), ())),
                              precision=_HI, preferred_element_type=_F32)
              + b1_ref[0])
    y = lax.dot_general(h, W2_ref[0], (((1,), (0,)), ((), ())),
                        precision=_HI, preferred_element_type=_F32) + b2_ref[0]
    y = jnp.where(eid_ref[...] == j, y, 0.0)
    prev = jnp.where(j == 0, 0.0, out_ref[...])
    out_ref[...] = prev + y


def kernel(x, Wr, br, gamma, beta, W1, b1, W2, b2):
    gamma3 = gamma.reshape(E, 1, D)
    beta3 = beta.reshape(E, 1, D)
    b13 = b1.reshape(E, 1, H)
    b23 = b2.reshape(E, 1, C)
    br2 = br.reshape(1, E)

    out = pl.pallas_call(
        _dense_body,
        grid=(E,),
        in_specs=[
            pl.BlockSpec((B, D), lambda j: (0, 0)),
            pl.BlockSpec((E, D), lambda j: (0, 0)),
            pl.BlockSpec((1, E), lambda j: (0, 0)),
            pl.BlockSpec((1, 1, D), lambda j: (j, 0, 0)),
            pl.BlockSpec((1, 1, D), lambda j: (j, 0, 0)),
            pl.BlockSpec((1, D, H), lambda j: (j, 0, 0)),
            pl.BlockSpec((1, 1, H), lambda j: (j, 0, 0)),
            pl.BlockSpec((1, H, C), lambda j: (j, 0, 0)),
            pl.BlockSpec((1, 1, C), lambda j: (j, 0, 0)),
        ],
        out_specs=pl.BlockSpec((B, C), lambda j: (0, 0)),
        out_shape=jax.ShapeDtypeStruct((B, C), _F32),
        scratch_shapes=[
            pltpu.VMEM((B, 1), jnp.int32),
            pltpu.VMEM((B, D), _F32),
        ],
    )(x, Wr, br2, gamma3, beta3, W1, b13, W2, b23)
    return out
